# Initial kernel scaffold; baseline (speedup 1.0000x reference)
import jax
import jax.numpy as jnp
from jax.experimental import pallas as pl


def kernel(x, edge_index, edge_attr, u, batch, e0_W, e0_b, n0a_W, n0a_b, n0b_W, n0b_b, g0_W, g0_b, e1_W, e1_b, n1a_W, n1a_b, n1b_W, n1b_b, g1_W, g1_b, bnx_g, bnx_b, bne_g, bne_b, bnu_g, bnu_b):
    raise NotImplementedError("write your pallas kernel here")


# speedup vs baseline: 2.7115x; 2.7115x over previous
import functools

import jax
import jax.numpy as jnp
from jax import lax
from jax.experimental import pallas as pl
from jax.experimental.pallas import tpu as pltpu
from jax.experimental.pallas import tpu_sc as plsc

_NC = 2
_NS = 16
_NW = _NC * _NS


def _mm_body(a_ref, w_ref, b_ref, o_ref, *, relu_out):
    acc = jnp.dot(a_ref[...], w_ref[...], preferred_element_type=jnp.float32)
    acc = acc + b_ref[...]
    if relu_out:
        acc = jnp.maximum(acc, 0.0)
    o_ref[...] = acc


def _mm_add_body(a_ref, w_ref, b_ref, add_ref, o_ref, *, relu_out):
    acc = jnp.dot(a_ref[...], w_ref[...], preferred_element_type=jnp.float32)
    acc = acc + b_ref[...] + add_ref[...]
    if relu_out:
        acc = jnp.maximum(acc, 0.0)
    o_ref[...] = acc


def _pick_bm(m, cap=2048):
    best = m
    for bm in range(8, cap + 1, 8):
        if m % bm == 0:
            best = bm
    return best if m % 8 == 0 or best != m else m


def _mm(a, w, bias=None, add=None, relu_out=False):
    m, k = a.shape
    ho = w.shape[1]
    if bias is None:
        bias = jnp.zeros((ho,), jnp.float32)
    bias2 = bias.reshape(1, ho)
    bm = _pick_bm(m)
    grid = (m // bm,)
    in_specs = [
        pl.BlockSpec((bm, k), lambda i: (i, 0)),
        pl.BlockSpec((k, ho), lambda i: (0, 0)),
        pl.BlockSpec((1, ho), lambda i: (0, 0)),
    ]
    args = [a, w, bias2]
    if add is not None:
        in_specs.append(pl.BlockSpec((bm, ho), lambda i: (i, 0)))
        args.append(add)
        body = functools.partial(_mm_add_body, relu_out=relu_out)
    else:
        body = functools.partial(_mm_body, relu_out=relu_out)
    return pl.pallas_call(
        body,
        grid=grid,
        in_specs=in_specs,
        out_specs=pl.BlockSpec((bm, ho), lambda i: (i, 0)),
        out_shape=jax.ShapeDtypeStruct((m, ho), jnp.float32),
    )(*args)


def _stats_body(x_ref, s_ref, q_ref):
    @pl.when(pl.program_id(0) == 0)
    def _():
        s_ref[...] = jnp.zeros_like(s_ref)
        q_ref[...] = jnp.zeros_like(q_ref)

    x = x_ref[...]
    s_ref[...] += jnp.sum(x, axis=0, keepdims=True)
    q_ref[...] += jnp.sum(x * x, axis=0, keepdims=True)


def _col_stats(x):
    m, h = x.shape
    bm = _pick_bm(m)
    s, q = pl.pallas_call(
        _stats_body,
        grid=(m // bm,),
        in_specs=[pl.BlockSpec((bm, h), lambda i: (i, 0))],
        out_specs=[pl.BlockSpec((1, h), lambda i: (0, 0))] * 2,
        out_shape=[jax.ShapeDtypeStruct((1, h), jnp.float32)] * 2,
    )(x)
    mean = s / m
    var = q / m - mean * mean
    return mean, var


def _bn(v, g, b):
    mean, var = _col_stats(v)
    return (v - mean) * (g / jnp.sqrt(var + 1e-5)) + b


def _sc_mesh():
    return plsc.VectorSubcoreMesh(
        core_axis_name="c", subcore_axis_name="s",
        num_cores=_NC, num_subcores=_NS)


def _gather_rows(table, idx):
    e = idx.shape[0]
    t, w = table.shape
    per = e // _NW
    chunk = 80
    n_chunks = per // chunk

    def body(table_hbm, idx_hbm, out_hbm, idx_v, rows_v, sem):
        wid = lax.axis_index("s") * _NC + lax.axis_index("c")
        base = wid * per

        def step(j, carry):
            off = base + j * chunk
            pltpu.sync_copy(idx_hbm.at[pl.ds(off, chunk)], idx_v)
            pltpu.async_copy(table_hbm.at[idx_v], rows_v, sem).wait()
            pltpu.sync_copy(rows_v, out_hbm.at[pl.ds(off, chunk)])
            return carry

        lax.fori_loop(0, n_chunks, step, 0)

    f = pl.kernel(
        body,
        out_type=jax.ShapeDtypeStruct((e, w), jnp.float32),
        mesh=_sc_mesh(),
        scratch_types=[
            pltpu.VMEM((chunk,), jnp.int32),
            pltpu.VMEM((chunk, w), jnp.float32),
            pltpu.SemaphoreType.DMA,
        ],
    )
    return f(table, idx)


def _scatter_add(vals, idx, t):
    e, w = vals.shape
    per = e // _NW
    chunk = 80
    n_chunks = per // chunk
    t_pad = -(-t // (_NS * 8)) * (_NS * 8)
    rpt = t_pad // _NS

    def body(vals_hbm, idx_hbm, zeros_hbm, out_hbm, idx_v, rows_v, acc_sh):
        cid = lax.axis_index("c")
        sid = lax.axis_index("s")
        wid = sid * _NC + cid
        base = wid * per
        pltpu.sync_copy(zeros_hbm.at[pl.ds(sid * rpt, rpt)],
                        acc_sh.at[pl.ds(sid * rpt, rpt)])
        plsc.subcore_barrier()

        def step(j, carry):
            off = base + j * chunk
            pltpu.sync_copy(idx_hbm.at[pl.ds(off, chunk)], idx_v)
            pltpu.sync_copy(vals_hbm.at[pl.ds(off, chunk)], rows_v)
            pltpu.sync_copy(rows_v, acc_sh.at[idx_v], add=True)
            return carry

        lax.fori_loop(0, n_chunks, step, 0)
        plsc.subcore_barrier()
        pltpu.sync_copy(acc_sh.at[pl.ds(sid * rpt, rpt)],
                        out_hbm.at[cid, pl.ds(sid * rpt, rpt)])

    f = pl.kernel(
        body,
        out_type=jax.ShapeDtypeStruct((_NC, t_pad, w), jnp.float32),
        mesh=_sc_mesh(),
        scratch_types=[
            pltpu.VMEM((chunk,), jnp.int32),
            pltpu.VMEM((chunk, w), jnp.float32),
            pltpu.VMEM_SHARED((t_pad, w), jnp.float32),
        ],
    )
    zeros = jnp.zeros((t_pad, w), jnp.float32)
    p = f(vals, idx, zeros)
    return p[0, :t] + p[1, :t]


def kernel(x, edge_index, edge_attr, u, batch,
           e0_W, e0_b, n0a_W, n0a_b, n0b_W, n0b_b, g0_W, g0_b,
           e1_W, e1_b, n1a_W, n1a_b, n1b_W, n1b_b, g1_W, g1_b,
           bnx_g, bnx_b, bne_g, bne_b, bnu_g, bnu_b):
    n, dx = x.shape
    g, du = u.shape
    h = e0_W.shape[1]
    de = edge_attr.shape[1]
    row = edge_index[0]
    col = edge_index[1]

    onehot_b = (batch[:, None] == jnp.arange(g, dtype=batch.dtype)[None, :])
    onehot_b = onehot_b.astype(jnp.float32)
    onehot_bt = onehot_b.T

    u0 = _mm(u, jnp.concatenate(
        [e0_W[2 * dx + de:], n0a_W[2 * dx + h:], n0b_W[dx + h:]], axis=1))
    ubn0 = _mm(onehot_b, u0)
    xw0 = _mm(x, jnp.concatenate(
        [e0_W[:dx], e0_W[dx:2 * dx], n0a_W[:dx], n0a_W[dx:2 * dx]], axis=1))
    trow0 = jnp.concatenate(
        [xw0[:, :h] + ubn0[:, :h], xw0[:, 3 * h:4 * h]], axis=1)
    tcol0 = jnp.concatenate(
        [xw0[:, h:2 * h], xw0[:, 2 * h:3 * h] + ubn0[:, h:2 * h]], axis=1)

    grow0 = _gather_rows(trow0, row)
    gcol0 = _gather_rows(tcol0, col)

    e0v = _mm(edge_attr, e0_W[2 * dx:2 * dx + de], bias=e0_b,
              add=grow0[:, :h] + gcol0[:, :h], relu_out=True)
    m0 = _mm(e0v, n0a_W[2 * dx:2 * dx + h], bias=n0a_b,
             add=grow0[:, h:] + gcol0[:, h:], relu_out=True)

    agg0 = _scatter_add(m0, col, n)
    x2 = _mm(jnp.concatenate([x, agg0], axis=1), n0b_W[:dx + h], bias=n0b_b,
             add=ubn0[:, 2 * h:], relu_out=True)

    agge0 = _scatter_add(e0v, col, n)
    ns0 = _mm(onehot_bt, x2)
    es0 = _mm(onehot_bt, agge0)
    u2 = _mm(jnp.concatenate([ns0, es0, u], axis=1), g0_W, bias=g0_b,
             relu_out=True)

    xb = _bn(x2, bnx_g, bnx_b)
    eb = _bn(e0v, bne_g, bne_b)
    ub = _bn(u2, bnu_g, bnu_b)

    u1 = _mm(ub, jnp.concatenate(
        [e1_W[3 * h:], n1a_W[3 * h:], n1b_W[2 * h:]], axis=1))
    ubn1 = _mm(onehot_b, u1)
    xw1 = _mm(xb, jnp.concatenate(
        [e1_W[:h], e1_W[h:2 * h], n1a_W[:h], n1a_W[h:2 * h]], axis=1))
    trow1 = jnp.concatenate(
        [xw1[:, :h] + ubn1[:, :h], xw1[:, 3 * h:4 * h]], axis=1)
    tcol1 = jnp.concatenate(
        [xw1[:, h:2 * h], xw1[:, 2 * h:3 * h] + ubn1[:, h:2 * h]], axis=1)

    grow1 = _gather_rows(trow1, row)
    gcol1 = _gather_rows(tcol1, col)

    e1v = _mm(eb, e1_W[2 * h:3 * h], bias=e1_b,
              add=grow1[:, :h] + gcol1[:, :h])
    m1 = _mm(e1v, n1a_W[2 * h:3 * h], bias=n1a_b,
             add=grow1[:, h:] + gcol1[:, h:])

    agg1 = _scatter_add(m1, col, n)
    x2_1 = _mm(jnp.concatenate([xb, agg1], axis=1), n1b_W[:2 * h],
               bias=n1b_b, add=ubn1[:, 2 * h:])

    agge1 = _scatter_add(e1v, col, n)
    ns1 = _mm(onehot_bt, x2_1)
    es1 = _mm(onehot_bt, agge1)
    u2_1 = _mm(jnp.concatenate([ns1, es1, ub], axis=1), g1_W, bias=g1_b)

    return (x2_1, e1v, u2_1)


# speedup vs baseline: 2.9611x; 1.0920x over previous
import functools

import jax
import jax.numpy as jnp
from jax import lax
from jax.experimental import pallas as pl
from jax.experimental.pallas import tpu as pltpu
from jax.experimental.pallas import tpu_sc as plsc

_NC = 2
_NS = 16
_NW = _NC * _NS


def _mm_body(a_ref, w_ref, b_ref, o_ref, *, relu_out):
    acc = jnp.dot(a_ref[...], w_ref[...], preferred_element_type=jnp.float32)
    acc = acc + b_ref[...]
    if relu_out:
        acc = jnp.maximum(acc, 0.0)
    o_ref[...] = acc


def _mm_add_body(a_ref, w_ref, b_ref, add_ref, o_ref, *, relu_out):
    acc = jnp.dot(a_ref[...], w_ref[...], preferred_element_type=jnp.float32)
    acc = acc + b_ref[...] + add_ref[...]
    if relu_out:
        acc = jnp.maximum(acc, 0.0)
    o_ref[...] = acc


def _mm_add2_body(a_ref, w_ref, b_ref, a1_ref, a2_ref, o_ref, *, relu_out):
    acc = jnp.dot(a_ref[...], w_ref[...], preferred_element_type=jnp.float32)
    acc = acc + b_ref[...] + (a1_ref[...] + a2_ref[...])
    if relu_out:
        acc = jnp.maximum(acc, 0.0)
    o_ref[...] = acc


def _pick_bm(m, cap=2048):
    best = m
    for bm in range(8, cap + 1, 8):
        if m % bm == 0:
            best = bm
    return best if m % 8 == 0 or best != m else m


def _mm(a, w, bias=None, adds=(), relu_out=False):
    m, k = a.shape
    ho = w.shape[1]
    if bias is None:
        bias = jnp.zeros((ho,), jnp.float32)
    bias2 = bias.reshape(1, ho)
    bm = _pick_bm(m)
    grid = (m // bm,)
    in_specs = [
        pl.BlockSpec((bm, k), lambda i: (i, 0)),
        pl.BlockSpec((k, ho), lambda i: (0, 0)),
        pl.BlockSpec((1, ho), lambda i: (0, 0)),
    ]
    args = [a, w, bias2]
    for arr, off in adds:
        in_specs.append(pl.BlockSpec((bm, ho), lambda i, _o=off: (i, _o)))
        args.append(arr)
    if len(adds) == 2:
        body = functools.partial(_mm_add2_body, relu_out=relu_out)
    elif len(adds) == 1:
        body = functools.partial(_mm_add_body, relu_out=relu_out)
    else:
        body = functools.partial(_mm_body, relu_out=relu_out)
    return pl.pallas_call(
        body,
        grid=grid,
        in_specs=in_specs,
        out_specs=pl.BlockSpec((bm, ho), lambda i: (i, 0)),
        out_shape=jax.ShapeDtypeStruct((m, ho), jnp.float32),
    )(*args)


def _sum_body(x_ref, s_ref):
    @pl.when(pl.program_id(0) == 0)
    def _():
        s_ref[...] = jnp.zeros_like(s_ref)

    s_ref[...] += jnp.sum(x_ref[...], axis=0, keepdims=True)


def _var_body(x_ref, m_ref, q_ref):
    @pl.when(pl.program_id(0) == 0)
    def _():
        q_ref[...] = jnp.zeros_like(q_ref)

    d = x_ref[...] - m_ref[...]
    q_ref[...] += jnp.sum(d * d, axis=0, keepdims=True)


def _col_stats(x):
    m, h = x.shape
    bm = _pick_bm(m)
    s = pl.pallas_call(
        _sum_body,
        grid=(m // bm,),
        in_specs=[pl.BlockSpec((bm, h), lambda i: (i, 0))],
        out_specs=pl.BlockSpec((1, h), lambda i: (0, 0)),
        out_shape=jax.ShapeDtypeStruct((1, h), jnp.float32),
    )(x)
    mean = s / m
    q = pl.pallas_call(
        _var_body,
        grid=(m // bm,),
        in_specs=[pl.BlockSpec((bm, h), lambda i: (i, 0)),
                  pl.BlockSpec((1, h), lambda i: (0, 0))],
        out_specs=pl.BlockSpec((1, h), lambda i: (0, 0)),
        out_shape=jax.ShapeDtypeStruct((1, h), jnp.float32),
    )(x, mean)
    var = q / m
    return mean, var


def _bn(v, g, b):
    mean, var = _col_stats(v)
    return (v - mean) * (g / jnp.sqrt(var + 1e-5)) + b


def _sc_mesh():
    return plsc.VectorSubcoreMesh(
        core_axis_name="c", subcore_axis_name="s",
        num_cores=_NC, num_subcores=_NS)


def _gather_rows(table, idx):
    e = idx.shape[0]
    t, w = table.shape
    per = e // _NW
    chunk = 80
    k = 5
    group = chunk * k
    n_groups = per // group

    def body(table_hbm, idx_hbm, out_hbm, idx_v, rows, sem_i, sem_g, sem_o):
        wid = lax.axis_index("s") * _NC + lax.axis_index("c")
        base = wid * per

        def step(gi, carry):
            off = base + gi * group
            ds_i = [pltpu.async_copy(
                idx_hbm.at[pl.ds(off + b * chunk, chunk)], idx_v.at[b], sem_i)
                for b in range(k)]
            for d in ds_i:
                d.wait()
            ds_g = [pltpu.async_copy(table_hbm.at[idx_v.at[b]], rows.at[b],
                                     sem_g) for b in range(k)]
            for d in ds_g:
                d.wait()
            ds_o = [pltpu.async_copy(rows.at[b],
                                     out_hbm.at[pl.ds(off + b * chunk, chunk)],
                                     sem_o) for b in range(k)]
            for d in ds_o:
                d.wait()
            return carry

        lax.fori_loop(0, n_groups, step, 0)

    f = pl.kernel(
        body,
        out_type=jax.ShapeDtypeStruct((e, w), jnp.float32),
        mesh=_sc_mesh(),
        scratch_types=[
            pltpu.VMEM((k, chunk), jnp.int32),
            pltpu.VMEM((k, chunk, w), jnp.float32),
            pltpu.SemaphoreType.DMA,
            pltpu.SemaphoreType.DMA,
            pltpu.SemaphoreType.DMA,
        ],
    )
    return f(table, idx)


def _scatter_add(vals, idx, t):
    e, w = vals.shape
    per = e // _NW
    chunk = 40
    t_pad = -(-t // (_NS * 8)) * (_NS * 8)
    rpt = t_pad // _NS

    k = 2
    group = chunk * k
    n_groups = per // group

    def body(vals_hbm, idx_hbm, zeros_hbm, out_hbm, *refs):
        idx_bufs = refs[:k]
        row_bufs = refs[k:2 * k]
        acc_sh = refs[2 * k]
        cid = lax.axis_index("c")
        sid = lax.axis_index("s")
        wid = sid * _NC + cid
        base = wid * per
        pltpu.sync_copy(zeros_hbm.at[pl.ds(sid * rpt, rpt)],
                        acc_sh.at[pl.ds(sid * rpt, rpt)])
        plsc.subcore_barrier()

        def step(gi, carry):
            off = base + gi * group
            for b in range(k):
                o = off + b * chunk
                pltpu.sync_copy(idx_hbm.at[pl.ds(o, chunk)], idx_bufs[b])
                pltpu.sync_copy(vals_hbm.at[pl.ds(o, chunk)], row_bufs[b])
            for b in range(k):
                pltpu.sync_copy(row_bufs[b], acc_sh.at[idx_bufs[b]], add=True)
            return carry

        lax.fori_loop(0, n_groups, step, 0)
        plsc.subcore_barrier()
        pltpu.sync_copy(acc_sh.at[pl.ds(sid * rpt, rpt)],
                        out_hbm.at[cid, pl.ds(sid * rpt, rpt)])

    f = pl.kernel(
        body,
        out_type=jax.ShapeDtypeStruct((_NC, t_pad, w), jnp.float32),
        mesh=_sc_mesh(),
        scratch_types=(
            [pltpu.VMEM((chunk,), jnp.int32) for _ in range(k)]
            + [pltpu.VMEM((chunk, w), jnp.float32) for _ in range(k)]
            + [pltpu.VMEM_SHARED((t_pad, w), jnp.float32)]
        ),
    )
    zeros = jnp.zeros((t_pad, w), jnp.float32)
    p = f(vals, idx, zeros)
    return p[0, :t] + p[1, :t]


def kernel(x, edge_index, edge_attr, u, batch,
           e0_W, e0_b, n0a_W, n0a_b, n0b_W, n0b_b, g0_W, g0_b,
           e1_W, e1_b, n1a_W, n1a_b, n1b_W, n1b_b, g1_W, g1_b,
           bnx_g, bnx_b, bne_g, bne_b, bnu_g, bnu_b):
    n, dx = x.shape
    g, du = u.shape
    h = e0_W.shape[1]
    de = edge_attr.shape[1]
    row = edge_index[0]
    col = edge_index[1]

    onehot_b = (batch[:, None] == jnp.arange(g, dtype=batch.dtype)[None, :])
    onehot_b = onehot_b.astype(jnp.float32)
    onehot_bt = onehot_b.T

    u0 = _mm(u, jnp.concatenate(
        [e0_W[2 * dx + de:], n0a_W[2 * dx + h:], n0b_W[dx + h:]], axis=1))
    ubn0 = _mm(onehot_b, u0)
    xw0 = _mm(x, jnp.concatenate(
        [e0_W[:dx], e0_W[dx:2 * dx], n0a_W[:dx], n0a_W[dx:2 * dx]], axis=1))
    trow0 = jnp.concatenate(
        [xw0[:, :h] + ubn0[:, :h], xw0[:, 3 * h:4 * h]], axis=1)
    tcol0 = jnp.concatenate(
        [xw0[:, h:2 * h], xw0[:, 2 * h:3 * h] + ubn0[:, h:2 * h]], axis=1)

    grow0 = _gather_rows(trow0, row)
    gcol0 = _gather_rows(tcol0, col)

    e0v = _mm(edge_attr, e0_W[2 * dx:2 * dx + de], bias=e0_b,
              adds=[(grow0, 0), (gcol0, 0)], relu_out=True)
    m0 = _mm(e0v, n0a_W[2 * dx:2 * dx + h], bias=n0a_b,
             adds=[(grow0, 1), (gcol0, 1)], relu_out=True)

    agg0 = _scatter_add(m0, col, n)
    x2 = _mm(jnp.concatenate([x, agg0], axis=1), n0b_W[:dx + h], bias=n0b_b,
             adds=[(ubn0, 2)], relu_out=True)

    agge0 = _scatter_add(e0v, col, n)
    ns0 = _mm(onehot_bt, x2)
    es0 = _mm(onehot_bt, agge0)
    u2 = _mm(jnp.concatenate([ns0, es0, u], axis=1), g0_W, bias=g0_b,
             relu_out=True)

    xb = _bn(x2, bnx_g, bnx_b)
    eb = _bn(e0v, bne_g, bne_b)
    ub = _bn(u2, bnu_g, bnu_b)

    u1 = _mm(ub, jnp.concatenate(
        [e1_W[3 * h:], n1a_W[3 * h:], n1b_W[2 * h:]], axis=1))
    ubn1 = _mm(onehot_b, u1)
    xw1 = _mm(xb, jnp.concatenate(
        [e1_W[:h], e1_W[h:2 * h], n1a_W[:h], n1a_W[h:2 * h]], axis=1))
    trow1 = jnp.concatenate(
        [xw1[:, :h] + ubn1[:, :h], xw1[:, 3 * h:4 * h]], axis=1)
    tcol1 = jnp.concatenate(
        [xw1[:, h:2 * h], xw1[:, 2 * h:3 * h] + ubn1[:, h:2 * h]], axis=1)

    grow1 = _gather_rows(trow1, row)
    gcol1 = _gather_rows(tcol1, col)

    e1v = _mm(eb, e1_W[2 * h:3 * h], bias=e1_b,
              adds=[(grow1, 0), (gcol1, 0)])
    m1 = _mm(e1v, n1a_W[2 * h:3 * h], bias=n1a_b,
             adds=[(grow1, 1), (gcol1, 1)])

    agg1 = _scatter_add(m1, col, n)
    x2_1 = _mm(jnp.concatenate([xb, agg1], axis=1), n1b_W[:2 * h],
               bias=n1b_b, adds=[(ubn1, 2)])

    agge1 = _scatter_add(e1v, col, n)
    ns1 = _mm(onehot_bt, x2_1)
    es1 = _mm(onehot_bt, agge1)
    u2_1 = _mm(jnp.concatenate([ns1, es1, ub], axis=1), g1_W, bias=g1_b)

    return (x2_1, e1v, u2_1)
